# Initial kernel scaffold; baseline (speedup 1.0000x reference)
import jax
import jax.numpy as jnp
from jax.experimental import pallas as pl


def kernel(x, edge_index, Wl1, Wr1, b1, Wl2, Wr2, b2, Wl3, Wr3, b3):
    raise NotImplementedError("write your pallas kernel here")


# speedup vs baseline: 7.0355x; 7.0355x over previous
import functools

import jax
import jax.numpy as jnp
from jax import lax
from jax.experimental import pallas as pl
from jax.experimental.pallas import tpu as pltpu
from jax.experimental.pallas import tpu_sc as plsc

N_NODES = 10000
N_PAD = 10240
D_IN = 128
D_HID = 64
W8 = 8
NC = 2
NS = 16
NW = NC * NS
CH = 128
BM = 1024
RPT = N_PAD // NS

_mesh = plsc.VectorSubcoreMesh(core_axis_name="c", subcore_axis_name="s")


def _seg_kernel(n_chunks, d, with_cnt):
    acc_type = jax.ShapeDtypeStruct((NC, N_PAD, d), jnp.float32)
    out_type = [acc_type] if with_cnt else acc_type
    scratch = [
        pltpu.VMEM((n_chunks, CH), jnp.int32),
        pltpu.VMEM((n_chunks, CH), jnp.int32),
        pltpu.VMEM((CH, d), jnp.float32),
        pltpu.VMEM_SHARED((N_PAD, d), jnp.float32),
        pltpu.SemaphoreType.DMA,
    ]
    if with_cnt:
        out_type.append(jax.ShapeDtypeStruct((NC, N_PAD, W8), jnp.float32))
        scratch += [
            pltpu.VMEM((CH, W8), jnp.float32),
            pltpu.VMEM_SHARED((N_PAD, W8), jnp.float32),
        ]

    def body(y_hbm, src_hbm, dst_hbm, z_hbm, z8_hbm, ones_hbm,
             acc_out, cnt_out, src_v, dst_v, rows_v, acc_sh, sem,
             ones_v=None, cnt_sh=None):
        c = lax.axis_index("c")
        s = lax.axis_index("s")
        wid = c * NS + s
        pltpu.sync_copy(src_hbm.at[wid], src_v)
        pltpu.sync_copy(dst_hbm.at[wid], dst_v)
        pltpu.sync_copy(z_hbm.at[pl.ds(s * RPT, RPT)],
                        acc_sh.at[pl.ds(s * RPT, RPT)])
        if with_cnt:
            pltpu.sync_copy(z8_hbm.at[pl.ds(s * RPT, RPT)],
                            cnt_sh.at[pl.ds(s * RPT, RPT)])
            pltpu.sync_copy(ones_hbm, ones_v)
        plsc.subcore_barrier()

        def chunk(j, carry):
            pltpu.async_copy(y_hbm.at[src_v.at[j]], rows_v, sem).wait()
            pltpu.sync_copy(rows_v, acc_sh.at[dst_v.at[j]], add=True)
            if with_cnt:
                pltpu.sync_copy(ones_v, cnt_sh.at[dst_v.at[j]], add=True)
            return carry

        lax.fori_loop(0, n_chunks, chunk, 0)
        plsc.subcore_barrier()
        pltpu.sync_copy(acc_sh.at[pl.ds(s * RPT, RPT)],
                        acc_out.at[c, pl.ds(s * RPT, RPT)])
        if with_cnt:
            pltpu.sync_copy(cnt_sh.at[pl.ds(s * RPT, RPT)],
                            cnt_out.at[c, pl.ds(s * RPT, RPT)])

    if with_cnt:
        def body_wrapped(y, srcr, dstr, z, z8, ones, acc_o, cnt_o,
                         src_v, dst_v, rows_v, acc_sh, sem, ones_v, cnt_sh):
            body(y, srcr, dstr, z, z8, ones, acc_o, cnt_o,
                 src_v, dst_v, rows_v, acc_sh, sem, ones_v, cnt_sh)
    else:
        def body_wrapped(y, srcr, dstr, z, z8, ones, acc_o,
                         src_v, dst_v, rows_v, acc_sh, sem):
            body(y, srcr, dstr, z, z8, ones, acc_o, None,
                 src_v, dst_v, rows_v, acc_sh, sem)

    return pl.kernel(body_wrapped, out_type=out_type, mesh=_mesh,
                     scratch_types=scratch,
                     compiler_params=pltpu.CompilerParams(
                         use_tc_tiling_on_sc=False))


def _mm_call(m_blk, k, n):
    def body(x_ref, w_ref, o_ref):
        o_ref[...] = jnp.dot(x_ref[...], w_ref[...],
                             preferred_element_type=jnp.float32)
    return pl.pallas_call(
        body,
        grid=(N_PAD // m_blk,),
        in_specs=[pl.BlockSpec((m_blk, k), lambda i: (i, 0)),
                  pl.BlockSpec((k, n), lambda i: (0, 0))],
        out_specs=pl.BlockSpec((m_blk, n), lambda i: (i, 0)),
        out_shape=jax.ShapeDtypeStruct((N_PAD, n), jnp.float32),
    )


def _upd_call(acc_w, hp_w, out_w, next_w, final):
    def body(*refs):
        if next_w is not None:
            a0, a1, c0, c1, hp, wr, b, wl, h_out, y_out = refs
        else:
            a0, a1, c0, c1, hp, wr, b, h_out = refs
        cnt = c0[:, 0:1] + c1[:, 0:1]
        inv = 1.0 / jnp.maximum(cnt, 1.0)
        agg = (a0[...] + a1[...]) * inv
        z = agg + jnp.dot(hp[...], wr[...],
                          preferred_element_type=jnp.float32) + b[...]
        if final:
            h = 1.0 / (1.0 + jnp.exp(-z))
        else:
            h = jnp.maximum(z, 0.0)
        h_out[...] = h
        if next_w is not None:
            y_out[...] = jnp.dot(h, wl[...], preferred_element_type=jnp.float32)

    in_specs = [
        pl.BlockSpec((BM, acc_w), lambda i: (i, 0)),
        pl.BlockSpec((BM, acc_w), lambda i: (i, 0)),
        pl.BlockSpec((BM, W8), lambda i: (i, 0)),
        pl.BlockSpec((BM, W8), lambda i: (i, 0)),
        pl.BlockSpec((BM, hp_w), lambda i: (i, 0)),
        pl.BlockSpec((hp_w, out_w), lambda i: (0, 0)),
        pl.BlockSpec((1, out_w), lambda i: (0, 0)),
    ]
    out_shape = jax.ShapeDtypeStruct((N_PAD, out_w), jnp.float32)
    out_specs = pl.BlockSpec((BM, out_w), lambda i: (i, 0))
    if next_w is not None:
        in_specs.append(pl.BlockSpec((out_w, next_w), lambda i: (0, 0)))
        out_shape = [out_shape,
                     jax.ShapeDtypeStruct((N_PAD, next_w), jnp.float32)]
        out_specs = [out_specs, pl.BlockSpec((BM, next_w), lambda i: (i, 0))]
    return pl.pallas_call(
        body, grid=(N_PAD // BM,),
        in_specs=in_specs, out_specs=out_specs, out_shape=out_shape,
    )


def kernel(x, edge_index, Wl1, Wr1, b1, Wl2, Wr2, b2, Wl3, Wr3, b3):
    e = edge_index.shape[1]
    n_chunks = -(-e // (NW * CH))
    e_pad = NW * n_chunks * CH
    src = edge_index[0].astype(jnp.int32)
    dst = edge_index[1].astype(jnp.int32)
    src = jnp.concatenate([src, jnp.zeros((e_pad - e,), jnp.int32)])
    dst = jnp.concatenate([dst, jnp.full((e_pad - e,), N_PAD - 1, jnp.int32)])
    srcw = src.reshape(NW, n_chunks, CH)
    dstw = dst.reshape(NW, n_chunks, CH)

    xp = jnp.pad(x, ((0, N_PAD - N_NODES), (0, 0)))
    z64 = jnp.zeros((N_PAD, D_HID), jnp.float32)
    z8 = jnp.zeros((N_PAD, W8), jnp.float32)
    ones8 = jnp.ones((CH, W8), jnp.float32)
    Wl3p = jnp.pad(Wl3, ((0, 0), (0, W8 - 1)))
    Wr3p = jnp.pad(Wr3, ((0, 0), (0, W8 - 1)))
    b3p = jnp.pad(b3, (0, W8 - 1)).reshape(1, W8)

    seg1 = _seg_kernel(n_chunks, D_HID, with_cnt=True)
    seg2 = _seg_kernel(n_chunks, D_HID, with_cnt=False)
    seg3 = _seg_kernel(n_chunks, W8, with_cnt=False)

    y1 = _mm_call(BM, D_IN, D_HID)(xp, Wl1)
    acc1, cnt = seg1(y1, srcw, dstw, z64, z8, ones8)
    h1, y2 = _upd_call(D_HID, D_IN, D_HID, D_HID, final=False)(
        acc1[0], acc1[1], cnt[0], cnt[1], xp, Wr1, b1.reshape(1, D_HID), Wl2)
    acc2 = seg2(y2, srcw, dstw, z64, z8, ones8)
    h2, y3 = _upd_call(D_HID, D_HID, D_HID, W8, final=False)(
        acc2[0], acc2[1], cnt[0], cnt[1], h1, Wr2, b2.reshape(1, D_HID), Wl3p)
    acc3 = seg3(y3, srcw, dstw, z8, z8, ones8)
    out8 = _upd_call(W8, D_HID, W8, None, final=True)(
        acc3[0], acc3[1], cnt[0], cnt[1], h2, Wr3p, b3p)
    return out8[:N_NODES, 0:1]
